# baseline (speedup 1.0000x reference)
import functools

import jax
import jax.numpy as jnp
from jax import lax
from jax.experimental import pallas as pl
from jax.experimental.pallas import tpu as pltpu
from jax.experimental.pallas import tpu_sc as plsc

_D = 64
_BLK = 4096
_NC, _NS = 2, 16
_NW = _NC * _NS
_F32 = jnp.float32


def _rup(n, m):
    return ((n + m - 1) // m) * m


def _pad_rows(x, m):
    p = (-x.shape[0]) % m
    if p:
        x = jnp.pad(x, ((0, p), (0, 0)))
    return x


def _mlp(x1, w1, b1, w3, b3, pre_adds=(), adds=(), out_t=False,
         out_also_t=False, out_halves=False):
    n0 = x1.shape[0]
    x1p = _pad_rows(x1, _BLK)
    npad = x1p.shape[0]
    grid = npad // _BLK
    dout = w3.shape[1]

    row_spec = lambda c: pl.BlockSpec((_BLK, c), lambda i: (i, 0))
    full_spec = lambda a: pl.BlockSpec(a.shape, lambda i: (0,) * a.ndim)

    args = [x1p, w1, b1.reshape(1, -1), w3, b3.reshape(1, -1)]
    specs = [row_spec(x1p.shape[1]), full_spec(w1),
             full_spec(b1.reshape(1, -1)), full_spec(w3),
             full_spec(b3.reshape(1, -1))]
    pre_p = tuple(_pad_rows(a, _BLK) for a in pre_adds)
    adds_p = tuple(_pad_rows(a, _BLK) for a in adds)
    for a in pre_p + adds_p:
        args.append(a)
        specs.append(row_spec(a.shape[1]))

    n_pre, n_post = len(pre_p), len(adds_p)

    def body(*refs):
        x1r, w1r, b1r, w3r, b3r = refs[:5]
        i = 5
        dmid = w1r.shape[1]
        pre = jnp.dot(x1r[...], w1r[...], preferred_element_type=_F32)
        for k in range(n_pre):
            pre = pre + refs[i + k][...][:, :dmid]
        i += n_pre
        h = jnp.maximum(pre + b1r[...], 0.0)
        o = jnp.dot(h, w3r[...], preferred_element_type=_F32) + b3r[...]
        for k in range(n_post):
            o = o + refs[i + k][...]
        i += n_post
        if out_halves:
            refs[i][...] = jnp.stack([o[:, :32], o[:, 32:]])
        elif out_t:
            refs[i][...] = o.T
        elif out_also_t:
            refs[i][...] = o
            refs[i + 1][...] = o.T
        else:
            refs[i][...] = o

    if out_halves:
        out_specs = pl.BlockSpec((2, _BLK, 32), lambda i: (0, i, 0))
        out_shape = jax.ShapeDtypeStruct((2, npad, 32), _F32)
    elif out_t:
        out_specs = pl.BlockSpec((dout, _BLK), lambda i: (0, i))
        out_shape = jax.ShapeDtypeStruct((dout, npad), _F32)
    elif out_also_t:
        out_specs = [pl.BlockSpec((_BLK, dout), lambda i: (i, 0)),
                     pl.BlockSpec((dout, _BLK), lambda i: (0, i))]
        out_shape = [jax.ShapeDtypeStruct((npad, dout), _F32),
                     jax.ShapeDtypeStruct((dout, npad), _F32)]
    else:
        out_specs = pl.BlockSpec((_BLK, dout), lambda i: (i, 0))
        out_shape = jax.ShapeDtypeStruct((npad, dout), _F32)

    out = pl.pallas_call(body, grid=(grid,), in_specs=specs,
                         out_specs=out_specs, out_shape=out_shape)(*args)
    return out


def _linear(x, w, b):
    n0 = x.shape[0]
    xp = _pad_rows(x, _BLK)
    grid = xp.shape[0] // _BLK
    dout = w.shape[1]

    def body(xr, wr, br, outr):
        outr[...] = jnp.dot(xr[...], wr[...],
                            preferred_element_type=_F32) + br[...]

    out = pl.pallas_call(
        body, grid=(grid,),
        in_specs=[pl.BlockSpec((_BLK, x.shape[1]), lambda i: (i, 0)),
                  pl.BlockSpec(w.shape, lambda i: (0, 0)),
                  pl.BlockSpec((1, dout), lambda i: (0, 0))],
        out_specs=pl.BlockSpec((_BLK, dout), lambda i: (i, 0)),
        out_shape=jax.ShapeDtypeStruct((xp.shape[0], dout), _F32))(
            xp, w, b.reshape(1, -1))
    return out


def _sc_rel(c128, src_p, dst_p):
    B = src_p.shape[0]
    rpw = B // _NW
    CH = 256
    nfull, tail = divmod(rpw, CH)

    scratch = [pltpu.VMEM((CH,), jnp.int32), pltpu.VMEM((CH,), jnp.int32),
               pltpu.VMEM((CH, 128), _F32), pltpu.VMEM((CH, 128), _F32),
               pltpu.VMEM((CH, 16), _F32),
               pltpu.SemaphoreType.DMA, pltpu.SemaphoreType.DMA]

    def body(tab_hbm, src_hbm, dst_hbm, out_hbm, si, di, sv, dv, ov,
             sem1, sem2):
        base = _wid() * rpw

        def chunk(off, sz):
            pltpu.sync_copy(src_hbm.at[pl.ds(off, sz)], si.at[pl.ds(0, sz)])
            pltpu.sync_copy(dst_hbm.at[pl.ds(off, sz)], di.at[pl.ds(0, sz)])
            cp1 = pltpu.async_copy(tab_hbm.at[si.at[pl.ds(0, sz)]],
                                   sv.at[pl.ds(0, sz)], sem1)
            cp2 = pltpu.async_copy(tab_hbm.at[di.at[pl.ds(0, sz)]],
                                   dv.at[pl.ds(0, sz)], sem2)
            cp1.wait()
            cp2.wait()

            def rloop(r, carry):
                ov[r, pl.ds(0, 16)] = sv[r, pl.ds(0, 16)] - dv[r, pl.ds(0, 16)]
                return carry

            lax.fori_loop(0, sz, rloop, 0)
            pltpu.sync_copy(ov.at[pl.ds(0, sz)], out_hbm.at[pl.ds(off, sz)])

        def loop(ci, carry):
            chunk(ci * CH + base, CH)
            return carry

        lax.fori_loop(0, nfull, loop, 0)
        if tail:
            chunk(base + nfull * CH, tail)

    return pl.kernel(
        body,
        out_type=jax.ShapeDtypeStruct((B, 16), _F32),
        mesh=plsc.VectorSubcoreMesh(**_MESH),
        scratch_types=scratch,
        compiler_params=pltpu.CompilerParams(needs_layout_passes=False),
    )(c128, src_p, dst_p)


def _untranspose(xT, bw):
    n = xT.shape[1]
    grid = n // bw

    def body(xr, outr):
        outr[...] = xr[...].T

    return pl.pallas_call(
        body, grid=(grid,),
        in_specs=[pl.BlockSpec((_D, bw), lambda i: (0, i))],
        out_specs=pl.BlockSpec((bw, _D), lambda i: (i, 0)),
        out_shape=jax.ShapeDtypeStruct((n, _D), _F32))(xT)


_MESH = dict(core_axis_name="c", subcore_axis_name="s")


def _wid():
    return lax.axis_index("s") * _NC + lax.axis_index("c")


def _sc_gather_rows(table, idx):
    V, Dm = table.shape
    B0 = idx.shape[0]
    Bp = _rup(B0, 2 * 392 * _NW)
    if Bp != B0:
        idx = jnp.pad(idx, (0, Bp - B0))
    rpw = Bp // _NW
    assert rpw % (2 * 392) == 0, rpw
    CH = 392
    nch = rpw // CH

    scratch = [pltpu.VMEM((CH,), jnp.int32), pltpu.VMEM((CH, Dm), _F32),
               pltpu.VMEM((CH,), jnp.int32), pltpu.VMEM((CH, Dm), _F32),
               pltpu.SemaphoreType.DMA, pltpu.SemaphoreType.DMA]

    def body(table_hbm, idx_hbm, out_hbm, i0, r0, i1, r1, sg, sw):
        base = _wid() * rpw

        def gather(ci, iv, rv):
            off = base + jnp.minimum(ci, nch - 1) * CH
            pltpu.sync_copy(idx_hbm.at[pl.ds(off, CH)], iv)
            pltpu.async_copy(table_hbm.at[iv], rv, sg).wait()

        def wb_start(ci, rv):
            return pltpu.async_copy(rv, out_hbm.at[pl.ds(base + ci * CH, CH)],
                                    sw)

        gather(0, i0, r0)

        def loop(ci2, carry):
            ci = ci2 * 2
            w0 = wb_start(ci, r0)
            gather(ci + 1, i1, r1)
            w0.wait()
            w1 = wb_start(ci + 1, r1)
            gather(ci + 2, i0, r0)
            w1.wait()
            return carry

        lax.fori_loop(0, nch // 2, loop, 0)

    return pl.kernel(body,
                     out_type=jax.ShapeDtypeStruct((Bp, Dm), _F32),
                     mesh=plsc.VectorSubcoreMesh(**_MESH),
                     scratch_types=scratch)(table, idx)


_SUB = 16


def _rmw_max_span(tabs, dst_v, updf, ngroups):
    def sloop(u, carry):
        base_g = u * _SUB

        def gfast(j, v):
            o = (base_g + j) * 16
            dstv = dst_v[pl.ds(o, 16)]
            for t, upd in zip(tabs, updf(o)):
                cur = plsc.load_gather(t, [dstv])
                new = jnp.maximum(cur, upd)
                plsc.store_scatter(t, [dstv], new)
                chk = plsc.load_gather(t, [dstv])
                v = v | (chk != new)
            return v

        viol = lax.fori_loop(0, _SUB, gfast, jnp.zeros((16,), jnp.bool_))

        @pl.when(jnp.any(viol))
        def _slow():
            def gslow(j, c):
                o = (base_g + j) * 16
                dstv = dst_v[pl.ds(o, 16)]
                for t, upd in zip(tabs, updf(o)):
                    cur = plsc.load_gather(t, [dstv])

                    def wbody(mm):
                        plsc.store_scatter(t, [dstv], upd, mask=mm)
                        return upd > plsc.load_gather(t, [dstv])

                    lax.while_loop(lambda mm: jnp.any(mm), wbody, upd > cur)
                return c

            lax.fori_loop(0, _SUB, gslow, 0)

        return carry

    lax.fori_loop(0, ngroups // _SUB, sloop, 0)


def _sc_scatter_max_T(hT, dst_p, n_out):
    Ep = dst_p.shape[0]
    W = _rup(n_out + 16, _BLK)
    CE = 2048
    nch = Ep // CE

    scratch = [pltpu.VMEM((W,), _F32), pltpu.VMEM((W,), _F32)]
    for _ in range(2):
        scratch += [pltpu.VMEM((CE,), jnp.int32),
                    pltpu.VMEM((CE,), _F32), pltpu.VMEM((CE,), _F32)]
    scratch += [pltpu.SemaphoreType.DMA, pltpu.SemaphoreType.DMA]

    def body(hT_hbm, dst_hbm, out_hbm, tab0, tab1,
             d0, a0, b0, d1, a1, b1, s0, s1):
        w = _wid()
        r0 = 2 * w
        z = jnp.zeros((16,), _F32)

        def zloop(i, carry):
            tab0[pl.ds(i * 16, 16)] = z
            tab1[pl.ds(i * 16, 16)] = z
            return carry

        lax.fori_loop(0, W // 16, zloop, 0)

        def start(ci, dv, av, bv, sem):
            off = jnp.minimum(ci * CE, Ep - CE)
            return (pltpu.async_copy(dst_hbm.at[pl.ds(off, CE)], dv, sem),
                    pltpu.async_copy(hT_hbm.at[r0, pl.ds(off, CE)], av, sem),
                    pltpu.async_copy(hT_hbm.at[r0 + 1, pl.ds(off, CE)],
                                     bv, sem))

        def compute(dv, av, bv):
            _rmw_max_span(
                (tab0, tab1), dv,
                lambda o: (av[pl.ds(o, 16)], bv[pl.ds(o, 16)]),
                CE // 16)

        for d in start(0, d0, a0, b0, s0):
            d.wait()

        def cloop(ci2, carry):
            ci = ci2 * 2
            ds = start(ci + 1, d1, a1, b1, s1)
            compute(d0, a0, b0)
            for d in ds:
                d.wait()
            ds2 = start(ci + 2, d0, a0, b0, s0)
            compute(d1, a1, b1)
            for d in ds2:
                d.wait()
            return carry

        lax.fori_loop(0, nch // 2, cloop, 0)
        pltpu.sync_copy(tab0, out_hbm.at[r0])
        pltpu.sync_copy(tab1, out_hbm.at[r0 + 1])

    return pl.kernel(
        body,
        out_type=jax.ShapeDtypeStruct((_D, W), _F32),
        mesh=plsc.VectorSubcoreMesh(**_MESH),
        scratch_types=scratch,
        compiler_params=pltpu.CompilerParams(needs_layout_passes=False),
    )(hT, dst_p)


def _sc_l2_block(t22T, lab_p, src_p, dst_p, n2):
    Ep_l = lab_p.shape[0]
    Ep_e = src_p.shape[0]
    W = n2 + 16
    CE = 4096

    scratch = [pltpu.VMEM((W,), _F32), pltpu.VMEM((W,), _F32),
               pltpu.VMEM((W,), _F32), pltpu.VMEM((W,), _F32),
               pltpu.VMEM((CE,), jnp.int32), pltpu.VMEM((CE,), jnp.int32),
               pltpu.VMEM((CE,), _F32), pltpu.VMEM((CE,), _F32)]

    def body(hT_hbm, lab_hbm, src_hbm, dst_hbm, out_hbm,
             t30, t31, ag0, ag1, ia_v, ib_v, h0_v, h1_v):
        w = _wid()
        r0 = 2 * w
        z = jnp.zeros((16,), _F32)

        def zloop(i, carry):
            t30[pl.ds(i * 16, 16)] = z
            t31[pl.ds(i * 16, 16)] = z
            ag0[pl.ds(i * 16, 16)] = z
            ag1[pl.ds(i * 16, 16)] = z
            return carry

        lax.fori_loop(0, W // 16, zloop, 0)

        def p1(ci, carry):
            off = ci * CE
            pltpu.sync_copy(lab_hbm.at[pl.ds(off, CE)], ia_v)
            pltpu.sync_copy(hT_hbm.at[r0, pl.ds(off, CE)], h0_v)
            pltpu.sync_copy(hT_hbm.at[r0 + 1, pl.ds(off, CE)], h1_v)
            _rmw_max_span(
                (t30, t31), ia_v,
                lambda o: (h0_v[pl.ds(o, 16)], h1_v[pl.ds(o, 16)]),
                CE // 16)
            return carry

        lax.fori_loop(0, Ep_l // CE, p1, 0)

        def p2(ci, carry):
            off = ci * CE
            pltpu.sync_copy(src_hbm.at[pl.ds(off, CE)], ia_v)
            pltpu.sync_copy(dst_hbm.at[pl.ds(off, CE)], ib_v)
            _rmw_max_span(
                (ag0, ag1), ib_v,
                lambda o: (plsc.load_gather(t30, [ia_v[pl.ds(o, 16)]]),
                           plsc.load_gather(t31, [ia_v[pl.ds(o, 16)]])),
                CE // 16)
            return carry

        lax.fori_loop(0, Ep_e // CE, p2, 0)

        def p3(i, carry):
            sl = pl.ds(i * 16, 16)
            ag0[sl] = ag0[sl] + t30[sl]
            ag1[sl] = ag1[sl] + t31[sl]
            return carry

        lax.fori_loop(0, W // 16, p3, 0)

        def p4(ci, carry):
            off = ci * CE
            pltpu.sync_copy(lab_hbm.at[pl.ds(off, CE)], ia_v)

            def g(j, c2):
                lv = ia_v[pl.ds(j * 16, 16)]
                h0_v[pl.ds(j * 16, 16)] = plsc.load_gather(ag0, [lv])
                h1_v[pl.ds(j * 16, 16)] = plsc.load_gather(ag1, [lv])
                return c2

            lax.fori_loop(0, CE // 16, g, 0)
            pltpu.sync_copy(h0_v, out_hbm.at[r0, pl.ds(off, CE)])
            pltpu.sync_copy(h1_v, out_hbm.at[r0 + 1, pl.ds(off, CE)])
            return carry

        lax.fori_loop(0, Ep_l // CE, p4, 0)

    return pl.kernel(
        body,
        out_type=jax.ShapeDtypeStruct((_D, Ep_l), _F32),
        mesh=plsc.VectorSubcoreMesh(**_MESH),
        scratch_types=scratch,
        compiler_params=pltpu.CompilerParams(needs_layout_passes=False),
    )(t22T, lab_p, src_p, dst_p)


def _sc_scatter_add(h_halves, lab_p, zeros_half, n_rows):
    Bp = lab_p.shape[0]
    Wr = zeros_half.shape[0]
    rps = Bp // _NS
    CH = min(512, rps)
    nfull, tail = divmod(rps, CH)
    wr_ps = Wr // _NS

    scratch = [pltpu.VMEM((CH,), jnp.int32), pltpu.VMEM((CH, 32), _F32),
               pltpu.VMEM_SHARED((Wr, 32), _F32)]
    if tail:
        scratch += [pltpu.VMEM((tail,), jnp.int32),
                    pltpu.VMEM((tail, 32), _F32)]

    def body(h_hbm, lab_hbm, z_hbm, out_hbm, idx_v, upd_v, shared, *ts):
        c = lax.axis_index("c")
        s = lax.axis_index("s")

        pltpu.sync_copy(z_hbm.at[pl.ds(s * wr_ps, wr_ps)],
                        shared.at[pl.ds(s * wr_ps, wr_ps)])
        plsc.subcore_barrier()

        base = s * rps

        def chunk(off, iv, uv, sz):
            pltpu.sync_copy(lab_hbm.at[pl.ds(off, sz)], iv)
            pltpu.sync_copy(h_hbm.at[c, pl.ds(off, sz)], uv)
            pltpu.sync_copy(uv, shared.at[iv], add=True)

        def loop(ci, carry):
            chunk(base + ci * CH, idx_v, upd_v, CH)
            return carry

        lax.fori_loop(0, nfull, loop, 0)
        if tail:
            chunk(base + nfull * CH, ts[0], ts[1], tail)

        plsc.subcore_barrier()
        pltpu.sync_copy(shared.at[pl.ds(s * wr_ps, wr_ps)],
                        out_hbm.at[c, pl.ds(s * wr_ps, wr_ps)])

    return pl.kernel(
        body,
        out_type=jax.ShapeDtypeStruct((2, Wr, 32), _F32),
        mesh=plsc.VectorSubcoreMesh(**_MESH),
        scratch_types=scratch,
        compiler_params=pltpu.CompilerParams(use_tc_tiling_on_sc=False),
    )(h_halves, lab_p, zeros_half)


def kernel(remission, points, l1_cluster_centers, l2_cluster_centers,
           l1_edges, l2_edges, l1_labels, l2_labels,
           Wf1, bf1, Wf2, bf2,
           We1, be1, We2, be2, Wo1, bo1, Wo2, bo2,
           Wc, bc):
    n1 = l1_cluster_centers.shape[0]
    n2 = l2_cluster_centers.shape[0]
    n_pts = remission.shape[0]
    E1 = l1_edges.shape[1]
    E2 = l2_edges.shape[1]
    z64 = jnp.zeros((_D,), _F32)

    def _pad_idx(a, total, base):
        p = total - a.shape[0]
        return jnp.concatenate(
            [a, base + (jnp.arange(p, dtype=jnp.int32) % 16)])

    src, dst = l1_edges[0], l1_edges[1]
    E1p = _rup(E1, _BLK)
    src_p = jnp.pad(src, (0, E1p - E1))
    dst_p = _pad_idx(dst, E1p, n1)

    We1a = jnp.pad(We1[:, :_D, :], ((0, 0), (0, 0), (0, 64)))
    be1p = jnp.pad(be1, ((0, 0), (0, 64)))
    We1b = jnp.pad(We1[:, _D:, :], ((0, 0), (0, 13), (0, 0)))
    z128 = jnp.zeros((128,), _F32)

    c128 = jnp.pad(l1_cluster_centers, ((0, 16), (0, 125)))
    rel16 = _sc_rel(c128, src_p, dst_p)

    Wf1p = jnp.pad(Wf1, ((0, 4), (0, 0)))
    Wneg = jnp.pad(-Wf1[1:4], ((0, 5), (0, 64)))
    c8 = jnp.pad(l1_cluster_centers, ((0, 0), (0, 5)))
    ccn = _linear(c8, Wneg, z128)
    ccn_g = _sc_gather_rows(ccn, l1_labels)
    p4 = jnp.pad(jnp.concatenate([remission, points], axis=1),
                 ((0, 0), (0, 4)))
    h0h = _mlp(p4, Wf1p, bf1, Wf2, bf2, pre_adds=(ccn_g,),
               out_halves=True)

    Bp = h0h.shape[1]
    lab1_p = _pad_idx(l1_labels, Bp, n1)
    Wr = _rup(n1 + 16, 128)
    zeros_half = jnp.zeros((Wr, 32), _F32)
    t1h = _sc_scatter_add(h0h, lab1_p, zeros_half, n1)
    feat = _pad_rows(
        jnp.concatenate([t1h[0, :n1], t1h[1, :n1]], axis=1), _BLK)

    def gnn(feat, i, adds=(), also_t=False):
        g = _linear(feat, We1a[i], be1p[i])
        gs = _sc_gather_rows(g, src_p)
        hT = _mlp(rel16, We1b[i], z64, We2[i], be2[i],
                  pre_adds=(gs,), out_t=True)
        aggT = _sc_scatter_max_T(hT, dst_p, n1)
        agg = _untranspose(aggT, _BLK)
        return _mlp(agg, Wo1[i], bo1[i], Wo2[i], bo2[i],
                    adds=(feat,) + adds, out_also_t=also_t)

    t2 = gnn(feat, 0)
    t2_1 = gnn(t2, 1)
    t2_2, t22T = gnn(t2_1, 2, also_t=True)

    Lp = t22T.shape[1]
    lab2_p = _pad_idx(l2_labels, Lp, n2)
    E2p = _rup(E2, _BLK)
    src2_p = jnp.pad(l2_edges[0], (0, E2p - E2))
    dst2_p = _pad_idx(l2_edges[1], E2p, n2)
    t5T = _sc_l2_block(t22T, lab2_p, src2_p, dst2_p, n2)
    t5 = _untranspose(t5T, _BLK)

    t6 = gnn(t5, 3, adds=(t2_2,))
    t6 = gnn(t6, 4, adds=(t2_1,))
    t6 = gnn(t6, 5, adds=(t2,))

    ncls = Wc.shape[1]
    Wcp = jnp.pad(Wc, ((0, 0), (0, 128 - ncls)))
    bcp = jnp.pad(bc, (0, 128 - ncls))
    logit_tab = _linear(t6, Wcp, bcp)
    lg = _sc_gather_rows(logit_tab, l1_labels)
    return lg[:n_pts, :ncls]


# speedup vs baseline: 1.0066x; 1.0066x over previous
import functools

import jax
import jax.numpy as jnp
from jax import lax
from jax.experimental import pallas as pl
from jax.experimental.pallas import tpu as pltpu
from jax.experimental.pallas import tpu_sc as plsc

_D = 64
_BLK = 4096
_NC, _NS = 2, 16
_NW = _NC * _NS
_F32 = jnp.float32


def _rup(n, m):
    return ((n + m - 1) // m) * m


def _pad_rows(x, m):
    p = (-x.shape[0]) % m
    if p:
        x = jnp.pad(x, ((0, p), (0, 0)))
    return x


def _mlp(x1, w1, b1, w3, b3, pre_adds=(), adds=(), out_t=False,
         out_also_t=False, out_halves=False, x1_t=False):
    if x1_t:
        npad = x1.shape[1]
        x1p = x1
    else:
        x1p = _pad_rows(x1, _BLK)
        npad = x1p.shape[0]
    grid = npad // _BLK
    dout = w3.shape[1]

    row_spec = lambda c: pl.BlockSpec((_BLK, c), lambda i: (i, 0))
    full_spec = lambda a: pl.BlockSpec(a.shape, lambda i: (0,) * a.ndim)

    x1_spec = (pl.BlockSpec((x1.shape[0], _BLK), lambda i: (0, i))
               if x1_t else row_spec(x1p.shape[1]))
    args = [x1p, w1, b1.reshape(1, -1), w3, b3.reshape(1, -1)]
    specs = [x1_spec, full_spec(w1),
             full_spec(b1.reshape(1, -1)), full_spec(w3),
             full_spec(b3.reshape(1, -1))]
    pre_p = tuple(_pad_rows(a, _BLK) for a in pre_adds)
    adds_p = tuple(_pad_rows(a, _BLK) for a in adds)
    for a in pre_p + adds_p:
        args.append(a)
        specs.append(row_spec(a.shape[1]))

    n_pre, n_post = len(pre_p), len(adds_p)

    def body(*refs):
        x1r, w1r, b1r, w3r, b3r = refs[:5]
        i = 5
        dmid = w1r.shape[1]
        x1v = x1r[...].T if x1_t else x1r[...]
        pre = jnp.dot(x1v, w1r[...], preferred_element_type=_F32)
        for k in range(n_pre):
            pre = pre + refs[i + k][...][:, :dmid]
        i += n_pre
        h = jnp.maximum(pre + b1r[...], 0.0)
        o = jnp.dot(h, w3r[...], preferred_element_type=_F32) + b3r[...]
        for k in range(n_post):
            o = o + refs[i + k][...]
        i += n_post
        if out_halves:
            refs[i][...] = jnp.stack([o[:, :32], o[:, 32:]])
        elif out_t:
            refs[i][...] = o.T
        elif out_also_t:
            refs[i][...] = o
            refs[i + 1][...] = o.T
        else:
            refs[i][...] = o

    if out_halves:
        out_specs = pl.BlockSpec((2, _BLK, 32), lambda i: (0, i, 0))
        out_shape = jax.ShapeDtypeStruct((2, npad, 32), _F32)
    elif out_t:
        out_specs = pl.BlockSpec((dout, _BLK), lambda i: (0, i))
        out_shape = jax.ShapeDtypeStruct((dout, npad), _F32)
    elif out_also_t:
        out_specs = [pl.BlockSpec((_BLK, dout), lambda i: (i, 0)),
                     pl.BlockSpec((dout, _BLK), lambda i: (0, i))]
        out_shape = [jax.ShapeDtypeStruct((npad, dout), _F32),
                     jax.ShapeDtypeStruct((dout, npad), _F32)]
    else:
        out_specs = pl.BlockSpec((_BLK, dout), lambda i: (i, 0))
        out_shape = jax.ShapeDtypeStruct((npad, dout), _F32)

    out = pl.pallas_call(body, grid=(grid,), in_specs=specs,
                         out_specs=out_specs, out_shape=out_shape)(*args)
    return out


def _linear(x, w, b):
    n0 = x.shape[0]
    xp = _pad_rows(x, _BLK)
    grid = xp.shape[0] // _BLK
    dout = w.shape[1]

    def body(xr, wr, br, outr):
        outr[...] = jnp.dot(xr[...], wr[...],
                            preferred_element_type=_F32) + br[...]

    out = pl.pallas_call(
        body, grid=(grid,),
        in_specs=[pl.BlockSpec((_BLK, x.shape[1]), lambda i: (i, 0)),
                  pl.BlockSpec(w.shape, lambda i: (0, 0)),
                  pl.BlockSpec((1, dout), lambda i: (0, 0))],
        out_specs=pl.BlockSpec((_BLK, dout), lambda i: (i, 0)),
        out_shape=jax.ShapeDtypeStruct((xp.shape[0], dout), _F32))(
            xp, w, b.reshape(1, -1))
    return out


def _sc_rel(c128, src_p, dst_p):
    B = src_p.shape[0]
    rpw = B // _NW
    CH = 256
    nfull, tail = divmod(rpw, CH)

    scratch = [pltpu.VMEM((CH,), jnp.int32), pltpu.VMEM((CH,), jnp.int32),
               pltpu.VMEM((CH, 128), _F32), pltpu.VMEM((CH, 128), _F32),
               pltpu.VMEM((CH, 16), _F32),
               pltpu.SemaphoreType.DMA, pltpu.SemaphoreType.DMA]

    def body(tab_hbm, src_hbm, dst_hbm, out_hbm, si, di, sv, dv, ov,
             sem1, sem2):
        base = _wid() * rpw

        def chunk(off, sz):
            pltpu.sync_copy(src_hbm.at[pl.ds(off, sz)], si.at[pl.ds(0, sz)])
            pltpu.sync_copy(dst_hbm.at[pl.ds(off, sz)], di.at[pl.ds(0, sz)])
            cp1 = pltpu.async_copy(tab_hbm.at[si.at[pl.ds(0, sz)]],
                                   sv.at[pl.ds(0, sz)], sem1)
            cp2 = pltpu.async_copy(tab_hbm.at[di.at[pl.ds(0, sz)]],
                                   dv.at[pl.ds(0, sz)], sem2)
            cp1.wait()
            cp2.wait()

            def rloop(r, carry):
                ov[r, pl.ds(0, 16)] = sv[r, pl.ds(0, 16)] - dv[r, pl.ds(0, 16)]
                return carry

            lax.fori_loop(0, sz, rloop, 0)
            pltpu.sync_copy(ov.at[pl.ds(0, sz)], out_hbm.at[pl.ds(off, sz)])

        def loop(ci, carry):
            chunk(ci * CH + base, CH)
            return carry

        lax.fori_loop(0, nfull, loop, 0)
        if tail:
            chunk(base + nfull * CH, tail)

    return pl.kernel(
        body,
        out_type=jax.ShapeDtypeStruct((B, 16), _F32),
        mesh=plsc.VectorSubcoreMesh(**_MESH),
        scratch_types=scratch,
        compiler_params=pltpu.CompilerParams(needs_layout_passes=False),
    )(c128, src_p, dst_p)


def _untranspose(xT, bw):
    n = xT.shape[1]
    grid = n // bw

    def body(xr, outr):
        outr[...] = xr[...].T

    return pl.pallas_call(
        body, grid=(grid,),
        in_specs=[pl.BlockSpec((_D, bw), lambda i: (0, i))],
        out_specs=pl.BlockSpec((bw, _D), lambda i: (i, 0)),
        out_shape=jax.ShapeDtypeStruct((n, _D), _F32))(xT)


_MESH = dict(core_axis_name="c", subcore_axis_name="s")


def _wid():
    return lax.axis_index("s") * _NC + lax.axis_index("c")


def _sc_gather_rows(table, idx):
    V, Dm = table.shape
    B0 = idx.shape[0]
    Bp = _rup(B0, 2 * 392 * _NW)
    if Bp != B0:
        idx = jnp.pad(idx, (0, Bp - B0))
    rpw = Bp // _NW
    assert rpw % (2 * 392) == 0, rpw
    CH = 392
    nch = rpw // CH

    scratch = [pltpu.VMEM((CH,), jnp.int32), pltpu.VMEM((CH, Dm), _F32),
               pltpu.VMEM((CH,), jnp.int32), pltpu.VMEM((CH, Dm), _F32),
               pltpu.SemaphoreType.DMA, pltpu.SemaphoreType.DMA]

    def body(table_hbm, idx_hbm, out_hbm, i0, r0, i1, r1, sg, sw):
        base = _wid() * rpw

        def gather(ci, iv, rv):
            off = base + jnp.minimum(ci, nch - 1) * CH
            pltpu.sync_copy(idx_hbm.at[pl.ds(off, CH)], iv)
            pltpu.async_copy(table_hbm.at[iv], rv, sg).wait()

        def wb_start(ci, rv):
            return pltpu.async_copy(rv, out_hbm.at[pl.ds(base + ci * CH, CH)],
                                    sw)

        gather(0, i0, r0)

        def loop(ci2, carry):
            ci = ci2 * 2
            w0 = wb_start(ci, r0)
            gather(ci + 1, i1, r1)
            w0.wait()
            w1 = wb_start(ci + 1, r1)
            gather(ci + 2, i0, r0)
            w1.wait()
            return carry

        lax.fori_loop(0, nch // 2, loop, 0)

    return pl.kernel(body,
                     out_type=jax.ShapeDtypeStruct((Bp, Dm), _F32),
                     mesh=plsc.VectorSubcoreMesh(**_MESH),
                     scratch_types=scratch)(table, idx)


_SUB = 16


def _rmw_max_span(tabs, dst_v, updf, ngroups):
    def sloop(u, carry):
        base_g = u * _SUB

        def gfast(j, v):
            o = (base_g + j) * 16
            dstv = dst_v[pl.ds(o, 16)]
            for t, upd in zip(tabs, updf(o)):
                cur = plsc.load_gather(t, [dstv])
                new = jnp.maximum(cur, upd)
                plsc.store_scatter(t, [dstv], new)
                chk = plsc.load_gather(t, [dstv])
                v = v | (chk != new)
            return v

        viol = lax.fori_loop(0, _SUB, gfast, jnp.zeros((16,), jnp.bool_))

        @pl.when(jnp.any(viol))
        def _slow():
            def gslow(j, c):
                o = (base_g + j) * 16
                dstv = dst_v[pl.ds(o, 16)]
                for t, upd in zip(tabs, updf(o)):
                    cur = plsc.load_gather(t, [dstv])

                    def wbody(mm):
                        plsc.store_scatter(t, [dstv], upd, mask=mm)
                        return upd > plsc.load_gather(t, [dstv])

                    lax.while_loop(lambda mm: jnp.any(mm), wbody, upd > cur)
                return c

            lax.fori_loop(0, _SUB, gslow, 0)

        return carry

    lax.fori_loop(0, ngroups // _SUB, sloop, 0)


def _sc_scatter_max_T(hT, dst_p, n_out):
    Ep = dst_p.shape[0]
    W = _rup(n_out + 16, _BLK)
    CE = 2048
    nch = Ep // CE

    scratch = [pltpu.VMEM((W,), _F32), pltpu.VMEM((W,), _F32)]
    for _ in range(2):
        scratch += [pltpu.VMEM((CE,), jnp.int32),
                    pltpu.VMEM((CE,), _F32), pltpu.VMEM((CE,), _F32)]
    scratch += [pltpu.SemaphoreType.DMA, pltpu.SemaphoreType.DMA]

    def body(hT_hbm, dst_hbm, out_hbm, tab0, tab1,
             d0, a0, b0, d1, a1, b1, s0, s1):
        w = _wid()
        r0 = 2 * w
        z = jnp.zeros((16,), _F32)

        def zloop(i, carry):
            tab0[pl.ds(i * 16, 16)] = z
            tab1[pl.ds(i * 16, 16)] = z
            return carry

        lax.fori_loop(0, W // 16, zloop, 0)

        def start(ci, dv, av, bv, sem):
            off = jnp.minimum(ci * CE, Ep - CE)
            return (pltpu.async_copy(dst_hbm.at[pl.ds(off, CE)], dv, sem),
                    pltpu.async_copy(hT_hbm.at[r0, pl.ds(off, CE)], av, sem),
                    pltpu.async_copy(hT_hbm.at[r0 + 1, pl.ds(off, CE)],
                                     bv, sem))

        def compute(dv, av, bv):
            _rmw_max_span(
                (tab0, tab1), dv,
                lambda o: (av[pl.ds(o, 16)], bv[pl.ds(o, 16)]),
                CE // 16)

        for d in start(0, d0, a0, b0, s0):
            d.wait()

        def cloop(ci2, carry):
            ci = ci2 * 2
            ds = start(ci + 1, d1, a1, b1, s1)
            compute(d0, a0, b0)
            for d in ds:
                d.wait()
            ds2 = start(ci + 2, d0, a0, b0, s0)
            compute(d1, a1, b1)
            for d in ds2:
                d.wait()
            return carry

        lax.fori_loop(0, nch // 2, cloop, 0)
        pltpu.sync_copy(tab0, out_hbm.at[r0])
        pltpu.sync_copy(tab1, out_hbm.at[r0 + 1])

    return pl.kernel(
        body,
        out_type=jax.ShapeDtypeStruct((_D, W), _F32),
        mesh=plsc.VectorSubcoreMesh(**_MESH),
        scratch_types=scratch,
        compiler_params=pltpu.CompilerParams(needs_layout_passes=False),
    )(hT, dst_p)


def _sc_l2_block(t22T, lab_p, src_p, dst_p, n2):
    Ep_l = lab_p.shape[0]
    Ep_e = src_p.shape[0]
    W = n2 + 16
    CE = 4096

    scratch = [pltpu.VMEM((W,), _F32), pltpu.VMEM((W,), _F32),
               pltpu.VMEM((W,), _F32), pltpu.VMEM((W,), _F32),
               pltpu.VMEM((CE,), jnp.int32), pltpu.VMEM((CE,), jnp.int32),
               pltpu.VMEM((CE,), _F32), pltpu.VMEM((CE,), _F32)]

    def body(hT_hbm, lab_hbm, src_hbm, dst_hbm, out_hbm,
             t30, t31, ag0, ag1, ia_v, ib_v, h0_v, h1_v):
        w = _wid()
        r0 = 2 * w
        z = jnp.zeros((16,), _F32)

        def zloop(i, carry):
            t30[pl.ds(i * 16, 16)] = z
            t31[pl.ds(i * 16, 16)] = z
            ag0[pl.ds(i * 16, 16)] = z
            ag1[pl.ds(i * 16, 16)] = z
            return carry

        lax.fori_loop(0, W // 16, zloop, 0)

        def p1(ci, carry):
            off = ci * CE
            pltpu.sync_copy(lab_hbm.at[pl.ds(off, CE)], ia_v)
            pltpu.sync_copy(hT_hbm.at[r0, pl.ds(off, CE)], h0_v)
            pltpu.sync_copy(hT_hbm.at[r0 + 1, pl.ds(off, CE)], h1_v)
            _rmw_max_span(
                (t30, t31), ia_v,
                lambda o: (h0_v[pl.ds(o, 16)], h1_v[pl.ds(o, 16)]),
                CE // 16)
            return carry

        lax.fori_loop(0, Ep_l // CE, p1, 0)

        def p2(ci, carry):
            off = ci * CE
            pltpu.sync_copy(src_hbm.at[pl.ds(off, CE)], ia_v)
            pltpu.sync_copy(dst_hbm.at[pl.ds(off, CE)], ib_v)
            _rmw_max_span(
                (ag0, ag1), ib_v,
                lambda o: (plsc.load_gather(t30, [ia_v[pl.ds(o, 16)]]),
                           plsc.load_gather(t31, [ia_v[pl.ds(o, 16)]])),
                CE // 16)
            return carry

        lax.fori_loop(0, Ep_e // CE, p2, 0)

        def p3(i, carry):
            sl = pl.ds(i * 16, 16)
            ag0[sl] = ag0[sl] + t30[sl]
            ag1[sl] = ag1[sl] + t31[sl]
            return carry

        lax.fori_loop(0, W // 16, p3, 0)

        def p4(ci, carry):
            off = ci * CE
            pltpu.sync_copy(lab_hbm.at[pl.ds(off, CE)], ia_v)

            def g(j, c2):
                lv = ia_v[pl.ds(j * 16, 16)]
                h0_v[pl.ds(j * 16, 16)] = plsc.load_gather(ag0, [lv])
                h1_v[pl.ds(j * 16, 16)] = plsc.load_gather(ag1, [lv])
                return c2

            lax.fori_loop(0, CE // 16, g, 0)
            pltpu.sync_copy(h0_v, out_hbm.at[r0, pl.ds(off, CE)])
            pltpu.sync_copy(h1_v, out_hbm.at[r0 + 1, pl.ds(off, CE)])
            return carry

        lax.fori_loop(0, Ep_l // CE, p4, 0)

    return pl.kernel(
        body,
        out_type=jax.ShapeDtypeStruct((_D, Ep_l), _F32),
        mesh=plsc.VectorSubcoreMesh(**_MESH),
        scratch_types=scratch,
        compiler_params=pltpu.CompilerParams(needs_layout_passes=False),
    )(t22T, lab_p, src_p, dst_p)


def _sc_scatter_add(h_halves, lab_p, zeros_half, n_rows):
    Bp = lab_p.shape[0]
    Wr = zeros_half.shape[0]
    rps = Bp // _NS
    CH = min(512, rps)
    nfull, tail = divmod(rps, CH)
    wr_ps = Wr // _NS

    scratch = [pltpu.VMEM((CH,), jnp.int32), pltpu.VMEM((CH, 32), _F32),
               pltpu.VMEM_SHARED((Wr, 32), _F32)]
    if tail:
        scratch += [pltpu.VMEM((tail,), jnp.int32),
                    pltpu.VMEM((tail, 32), _F32)]

    def body(h_hbm, lab_hbm, z_hbm, out_hbm, idx_v, upd_v, shared, *ts):
        c = lax.axis_index("c")
        s = lax.axis_index("s")

        pltpu.sync_copy(z_hbm.at[pl.ds(s * wr_ps, wr_ps)],
                        shared.at[pl.ds(s * wr_ps, wr_ps)])
        plsc.subcore_barrier()

        base = s * rps

        def chunk(off, iv, uv, sz):
            pltpu.sync_copy(lab_hbm.at[pl.ds(off, sz)], iv)
            pltpu.sync_copy(h_hbm.at[c, pl.ds(off, sz)], uv)
            pltpu.sync_copy(uv, shared.at[iv], add=True)

        def loop(ci, carry):
            chunk(base + ci * CH, idx_v, upd_v, CH)
            return carry

        lax.fori_loop(0, nfull, loop, 0)
        if tail:
            chunk(base + nfull * CH, ts[0], ts[1], tail)

        plsc.subcore_barrier()
        pltpu.sync_copy(shared.at[pl.ds(s * wr_ps, wr_ps)],
                        out_hbm.at[c, pl.ds(s * wr_ps, wr_ps)])

    return pl.kernel(
        body,
        out_type=jax.ShapeDtypeStruct((2, Wr, 32), _F32),
        mesh=plsc.VectorSubcoreMesh(**_MESH),
        scratch_types=scratch,
        compiler_params=pltpu.CompilerParams(use_tc_tiling_on_sc=False),
    )(h_halves, lab_p, zeros_half)


def kernel(remission, points, l1_cluster_centers, l2_cluster_centers,
           l1_edges, l2_edges, l1_labels, l2_labels,
           Wf1, bf1, Wf2, bf2,
           We1, be1, We2, be2, Wo1, bo1, Wo2, bo2,
           Wc, bc):
    n1 = l1_cluster_centers.shape[0]
    n2 = l2_cluster_centers.shape[0]
    n_pts = remission.shape[0]
    E1 = l1_edges.shape[1]
    E2 = l2_edges.shape[1]
    z64 = jnp.zeros((_D,), _F32)

    def _pad_idx(a, total, base):
        p = total - a.shape[0]
        return jnp.concatenate(
            [a, base + (jnp.arange(p, dtype=jnp.int32) % 16)])

    src, dst = l1_edges[0], l1_edges[1]
    E1p = _rup(E1, _BLK)
    src_p = jnp.pad(src, (0, E1p - E1))
    dst_p = _pad_idx(dst, E1p, n1)

    We1a = jnp.pad(We1[:, :_D, :], ((0, 0), (0, 0), (0, 64)))
    be1p = jnp.pad(be1, ((0, 0), (0, 64)))
    We1b = jnp.pad(We1[:, _D:, :], ((0, 0), (0, 13), (0, 0)))
    z128 = jnp.zeros((128,), _F32)

    c128 = jnp.pad(l1_cluster_centers, ((0, 16), (0, 125)))
    rel16 = _sc_rel(c128, src_p, dst_p)

    Wf1p = jnp.pad(Wf1, ((0, 4), (0, 0)))
    Wneg = jnp.pad(-Wf1[1:4], ((0, 5), (0, 64)))
    c8 = jnp.pad(l1_cluster_centers, ((0, 0), (0, 5)))
    ccn = _linear(c8, Wneg, z128)
    ccn_g = _sc_gather_rows(ccn, l1_labels)
    p4 = jnp.pad(jnp.concatenate([remission, points], axis=1),
                 ((0, 0), (0, 4)))
    h0h = _mlp(p4, Wf1p, bf1, Wf2, bf2, pre_adds=(ccn_g,),
               out_halves=True)

    Bp = h0h.shape[1]
    lab1_p = _pad_idx(l1_labels, Bp, n1)
    Wr = _rup(n1 + 16, 128)
    zeros_half = jnp.zeros((Wr, 32), _F32)
    t1h = _sc_scatter_add(h0h, lab1_p, zeros_half, n1)
    feat = _pad_rows(
        jnp.concatenate([t1h[0, :n1], t1h[1, :n1]], axis=1), _BLK)

    def gnn(feat, i, adds=(), also_t=False):
        g = _linear(feat, We1a[i], be1p[i])
        gs = _sc_gather_rows(g, src_p)
        hT = _mlp(rel16, We1b[i], z64, We2[i], be2[i],
                  pre_adds=(gs,), out_t=True)
        aggT = _sc_scatter_max_T(hT, dst_p, n1)
        return _mlp(aggT, Wo1[i], bo1[i], Wo2[i], bo2[i], x1_t=True,
                    adds=(feat,) + adds, out_also_t=also_t)

    t2 = gnn(feat, 0)
    t2_1 = gnn(t2, 1)
    t2_2, t22T = gnn(t2_1, 2, also_t=True)

    Lp = t22T.shape[1]
    lab2_p = _pad_idx(l2_labels, Lp, n2)
    E2p = _rup(E2, _BLK)
    src2_p = jnp.pad(l2_edges[0], (0, E2p - E2))
    dst2_p = _pad_idx(l2_edges[1], E2p, n2)
    t5T = _sc_l2_block(t22T, lab2_p, src2_p, dst2_p, n2)
    t5 = _untranspose(t5T, _BLK)

    t6 = gnn(t5, 3, adds=(t2_2,))
    t6 = gnn(t6, 4, adds=(t2_1,))
    t6 = gnn(t6, 5, adds=(t2,))

    ncls = Wc.shape[1]
    Wcp = jnp.pad(Wc, ((0, 0), (0, 128 - ncls)))
    bcp = jnp.pad(bc, (0, 128 - ncls))
    logit_tab = _linear(t6, Wcp, bcp)
    lg = _sc_gather_rows(logit_tab, l1_labels)
    return lg[:n_pts, :ncls]
